# Initial kernel scaffold; baseline (speedup 1.0000x reference)
import jax
import jax.numpy as jnp
from jax.experimental import pallas as pl


def kernel(site_type_map, node_size_x, node_size_y):
    raise NotImplementedError("write your pallas kernel here")


# speedup vs baseline: 935.0159x; 935.0159x over previous
import jax
import jax.numpy as jnp
from jax import lax
from jax.experimental import pallas as pl
from jax.experimental.pallas import tpu as pltpu

_W = 512
_H = 512
_NBX = 512
_NBY = 512
_BIN_AREA = ((512.0 - 0.0) / _NBX) * ((512.0 - 0.0) / _NBY)


def _wcoef(n):
    w0 = jnp.maximum(jnp.minimum(n, 1.0), 0.0)
    w1 = jnp.maximum(jnp.minimum(n - 1.0, 1.0), 0.0)
    return w0, w1


def _tc_body(nsx_ref, nsy_ref, site_ref, o1_ref, o2_ref, o3_ref):
    site = site_ref[...]
    zrow = jnp.zeros((1, _H), jnp.float32)
    zcol = jnp.zeros((_W, 1), jnp.float32)

    def cap(t, sx, sy, out_ref):
        w0, w1 = _wcoef(nsx_ref[sx])
        h0, h1 = _wcoef(nsy_ref[sy])
        m = (site == t).astype(jnp.float32)
        md = jnp.concatenate([zrow, m[:-1, :]], axis=0)
        mr = jnp.concatenate([zcol, m[:, :-1]], axis=1)
        mdr = jnp.concatenate([zcol, md[:, :-1]], axis=1)
        out_ref[...] = _BIN_AREA - (w0 * h0 * m + w1 * h0 * md
                                    + w0 * h1 * mr + w1 * h1 * mdr)

    cap(1, 0, 0, o1_ref)
    cap(2, 2, 2, o2_ref)
    cap(3, 3, 3, o3_ref)


def _tc_call(site_type_map, node_size_x, node_size_y, interpret=False):
    out = jax.ShapeDtypeStruct((_NBX, _NBY), jnp.float32)
    return pl.pallas_call(
        _tc_body,
        out_shape=(out, out, out),
        in_specs=[
            pl.BlockSpec(memory_space=pltpu.SMEM),
            pl.BlockSpec(memory_space=pltpu.SMEM),
            pl.BlockSpec(memory_space=pltpu.VMEM),
        ],
        out_specs=(
            pl.BlockSpec(memory_space=pltpu.VMEM),
            pl.BlockSpec(memory_space=pltpu.VMEM),
            pl.BlockSpec(memory_space=pltpu.VMEM),
        ),
        interpret=interpret,
    )(node_size_x, node_size_y, site_type_map)


def kernel(site_type_map, node_size_x, node_size_y):
    a, b, c = _tc_call(site_type_map, node_size_x, node_size_y)
    return (a, a, a, a, a, b, c)
